# baseline (speedup 1.0000x reference)
import functools

import jax
import jax.numpy as jnp
from jax import lax
from jax.experimental import pallas as pl
from jax.experimental.pallas import tpu as pltpu
from jax.experimental.pallas import tpu_sc as plsc

_CHUNK = 16
_NBUF = 10


@functools.lru_cache(maxsize=None)
def _make_gather(R, C, D, chunk, nbuf):
    B = R * C
    info = plsc.get_sparse_core_info()
    NC, NS = info.num_cores, info.num_subcores
    NW = NC * NS
    assert B % NW == 0
    b_per_w = B // NW
    assert C % b_per_w == 0
    assert b_per_w % chunk == 0
    n_chunks = b_per_w // chunk
    lookahead = 8
    mesh = plsc.VectorSubcoreMesh(core_axis_name="c", subcore_axis_name="s")

    @functools.partial(
        pl.kernel,
        mesh=mesh,
        out_type=jax.ShapeDtypeStruct((B, D), jnp.float32),
        scratch_types=[
            pltpu.VMEM((b_per_w,), jnp.int32),
            pltpu.VMEM((nbuf, chunk, D), jnp.float32),
        ]
        + [pltpu.SemaphoreType.DMA] * (2 * nbuf),
    )
    def k(table_hbm, tok_hbm, out_hbm, idx_v, buf, *sems):
        gsem = sems[:nbuf]
        psem = sems[nbuf:]
        wid = lax.axis_index("s") * NC + lax.axis_index("c")
        base = wid * b_per_w
        per_row = C // b_per_w
        trow = wid // per_row
        tcol = (wid % per_row) * b_per_w
        half = b_per_w // 2
        pltpu.sync_copy(
            tok_hbm.at[trow, pl.ds(tcol, half)], idx_v.at[pl.ds(0, half)]
        )

        gathers = [None] * n_chunks
        puts = [None] * n_chunks

        def start_gather(c):
            gathers[c] = pltpu.async_copy(
                table_hbm.at[idx_v.at[pl.ds(c * chunk, chunk)]],
                buf.at[c % nbuf],
                gsem[c % nbuf],
            )

        lead = min(lookahead, n_chunks, half // chunk)
        for c in range(lead):
            start_gather(c)
        pltpu.sync_copy(
            tok_hbm.at[trow, pl.ds(tcol + half, b_per_w - half)],
            idx_v.at[pl.ds(half, b_per_w - half)],
        )
        for c in range(lead, min(lookahead, n_chunks)):
            start_gather(c)
        for c in range(n_chunks):
            nxt = c + lookahead
            if nxt < n_chunks:
                if nxt >= nbuf:
                    puts[nxt - nbuf].wait()
                start_gather(nxt)
            gathers[c].wait()
            puts[c] = pltpu.async_copy(
                buf.at[c % nbuf],
                out_hbm.at[pl.ds(base + c * chunk, chunk)],
                psem[c % nbuf],
            )
        for c in range(max(0, n_chunks - nbuf), n_chunks):
            if puts[c] is not None:
                puts[c].wait()

    return k


def kernel(tokens, W_E):
    R, C = tokens.shape
    D = W_E.shape[1]
    out = _make_gather(R, C, D, _CHUNK, _NBUF)(W_E, tokens)
    return out.reshape(tokens.shape + (D,))


# speedup vs baseline: 1.0064x; 1.0064x over previous
import functools

import jax
import jax.numpy as jnp
from jax import lax
from jax.experimental import pallas as pl
from jax.experimental.pallas import tpu as pltpu
from jax.experimental.pallas import tpu_sc as plsc

_CHUNK = 32
_NBUF = 5


@functools.lru_cache(maxsize=None)
def _make_gather(R, C, D, chunk, nbuf):
    B = R * C
    info = plsc.get_sparse_core_info()
    NC, NS = info.num_cores, info.num_subcores
    NW = NC * NS
    assert B % NW == 0
    b_per_w = B // NW
    assert C % b_per_w == 0
    assert b_per_w % chunk == 0
    n_chunks = b_per_w // chunk
    lookahead = 4
    mesh = plsc.VectorSubcoreMesh(core_axis_name="c", subcore_axis_name="s")

    @functools.partial(
        pl.kernel,
        mesh=mesh,
        out_type=jax.ShapeDtypeStruct((B, D), jnp.float32),
        scratch_types=[
            pltpu.VMEM((b_per_w,), jnp.int32),
            pltpu.VMEM((nbuf, chunk, D), jnp.float32),
        ]
        + [pltpu.SemaphoreType.DMA] * (2 * nbuf),
    )
    def k(table_hbm, tok_hbm, out_hbm, idx_v, buf, *sems):
        gsem = sems[:nbuf]
        psem = sems[nbuf:]
        wid = lax.axis_index("s") * NC + lax.axis_index("c")
        base = wid * b_per_w
        per_row = C // b_per_w
        trow = wid // per_row
        tcol = (wid % per_row) * b_per_w
        half = b_per_w // 2
        pltpu.sync_copy(
            tok_hbm.at[trow, pl.ds(tcol, half)], idx_v.at[pl.ds(0, half)]
        )

        gathers = [None] * n_chunks
        puts = [None] * n_chunks

        def start_gather(c):
            gathers[c] = pltpu.async_copy(
                table_hbm.at[idx_v.at[pl.ds(c * chunk, chunk)]],
                buf.at[c % nbuf],
                gsem[c % nbuf],
            )

        lead = min(lookahead, n_chunks, half // chunk)
        for c in range(lead):
            start_gather(c)
        pltpu.sync_copy(
            tok_hbm.at[trow, pl.ds(tcol + half, b_per_w - half)],
            idx_v.at[pl.ds(half, b_per_w - half)],
        )
        for c in range(lead, min(lookahead, n_chunks)):
            start_gather(c)
        for c in range(n_chunks):
            nxt = c + lookahead
            if nxt < n_chunks:
                if nxt >= nbuf:
                    puts[nxt - nbuf].wait()
                start_gather(nxt)
            gathers[c].wait()
            puts[c] = pltpu.async_copy(
                buf.at[c % nbuf],
                out_hbm.at[pl.ds(base + c * chunk, chunk)],
                psem[c % nbuf],
            )
        for c in range(max(0, n_chunks - nbuf), n_chunks):
            if puts[c] is not None:
                puts[c].wait()

    return k


def kernel(tokens, W_E):
    R, C = tokens.shape
    D = W_E.shape[1]
    out = _make_gather(R, C, D, _CHUNK, _NBUF)(W_E, tokens)
    return out.reshape(tokens.shape + (D,))
